# baseline (speedup 1.0000x reference)
import functools

import jax
import jax.numpy as jnp
from jax import lax
from jax.experimental import pallas as pl
from jax.experimental.pallas import tpu as pltpu
from jax.experimental.pallas import tpu_sc as plsc

N = 10000
E = 160000
D = 256
HALF = 128
NC = 2
NS = 16
NP = 10240
EROWS = 1280
EPAD = EROWS * 128 - E
ROWS_A = EROWS // (NC * NS)
ROWS_C = EROWS // NS
OUT_RPT = NP // NS

_MESH = plsc.VectorSubcoreMesh(
    core_axis_name="c", subcore_axis_name="s", num_cores=NC, num_subcores=NS
)
_SC_PARAMS = pltpu.CompilerParams(needs_layout_passes=False)


SEG = 2 * NP // NS


def _deg_body(ei3, out_o, out_i, idx_s, idx_d, hist_o, hist_i, red_v,
              stage_s):
  c = lax.axis_index("c")
  s = lax.axis_index("s")
  wid = s * NC + c
  zero = jnp.zeros((16,), jnp.float32)

  @pl.loop(0, NP // 16)
  def _(i):
    hist_o[pl.ds(i * 16, 16)] = zero
    hist_i[pl.ds(i * 16, 16)] = zero

  ones = jnp.ones((16,), jnp.float32)

  pltpu.sync_copy(ei3.at[0, pl.ds(wid * ROWS_A, ROWS_A)], idx_s)
  pltpu.sync_copy(ei3.at[1, pl.ds(wid * ROWS_A, ROWS_A)], idx_d)

  @pl.loop(0, ROWS_A)
  def _(r):
    for k in range(8):
      vs = idx_s[r, pl.ds(k * 16, 16)]
      vd = idx_d[r, pl.ds(k * 16, 16)]
      plsc.addupdate_scatter(hist_o, [vs], ones)
      plsc.addupdate_scatter(hist_i, [vd], ones)

  pltpu.sync_copy(hist_o, stage_s.at[pl.ds((2 * s) * NP, NP)])
  pltpu.sync_copy(hist_i, stage_s.at[pl.ds((2 * s + 1) * NP, NP)])
  plsc.subcore_barrier()
  for t in range(NS):
    pltpu.sync_copy(stage_s.at[pl.ds(t * 2 * NP + s * SEG, SEG)], red_v.at[t])

  @pl.loop(0, SEG // 16)
  def _(j):
    acc = red_v[0, pl.ds(j * 16, 16)]
    for t in range(1, NS):
      acc = acc + red_v[t, pl.ds(j * 16, 16)]
    hist_o[pl.ds(j * 16, 16)] = acc

  @pl.when(s < 8)
  def _():
    pltpu.sync_copy(hist_o.at[pl.ds(0, SEG)],
                    out_o.at[pl.ds(c * NP + s * SEG, SEG)])

  @pl.when(s >= 8)
  def _():
    pltpu.sync_copy(hist_o.at[pl.ds(0, SEG)],
                    out_i.at[pl.ds(c * NP + (s - 8) * SEG, SEG)])


_deg_call = pl.kernel(
    _deg_body,
    out_type=[
        jax.ShapeDtypeStruct((NC * NP,), jnp.float32),
        jax.ShapeDtypeStruct((NC * NP,), jnp.float32),
    ],
    mesh=_MESH,
    scratch_types=[
        pltpu.VMEM((ROWS_A, 128), jnp.int32),
        pltpu.VMEM((ROWS_A, 128), jnp.int32),
        pltpu.VMEM((NP,), jnp.float32),
        pltpu.VMEM((NP,), jnp.float32),
        pltpu.VMEM((NS, SEG), jnp.float32),
        pltpu.VMEM_SHARED((NS * 2 * NP,), jnp.float32),
    ],
    compiler_params=_SC_PARAMS,
)


HROWS = ROWS_C // 2


def _agg_body(h_hbm, ei3, agg0, agg1, src_v, dst_v, rows0, rows1,
              agg_s, sem0, sem1):
  c = lax.axis_index("c")
  s = lax.axis_index("s")
  zero = jnp.zeros((16,), jnp.float32)

  @pl.loop(0, 128)
  def _(r):
    for k in range(8):
      rows0[r, pl.ds(k * 16, 16)] = zero

  for j in range(OUT_RPT // 128):
    pltpu.sync_copy(rows0, agg_s.at[pl.ds(s * OUT_RPT + j * 128, 128)])
  plsc.subcore_barrier()

  offv = jnp.broadcast_to(c * NP, (16,)).astype(jnp.int32)

  def _batch(base, nrows):
    pltpu.sync_copy(ei3.at[0, pl.ds(base, nrows)], src_v.at[pl.ds(0, nrows)])
    pltpu.sync_copy(ei3.at[1, pl.ds(base, nrows)], dst_v.at[pl.ds(0, nrows)])

    @pl.loop(0, nrows)
    def _(r):
      for k in range(8):
        src_v[r, pl.ds(k * 16, 16)] = src_v[r, pl.ds(k * 16, 16)] + offv

    pltpu.async_copy(h_hbm.at[src_v.at[0]], rows0, sem0)
    pltpu.async_copy(h_hbm.at[src_v.at[1]], rows1, sem1)

    @pl.loop(0, nrows, step=2)
    def _(r):
      pltpu.make_async_copy(h_hbm.at[src_v.at[0]], rows0, sem0).wait()
      pltpu.sync_copy(rows0, agg_s.at[dst_v.at[r]], add=True)

      @pl.when(r < nrows - 2)
      def _():
        pltpu.async_copy(h_hbm.at[src_v.at[r + 2]], rows0, sem0)

      pltpu.make_async_copy(h_hbm.at[src_v.at[1]], rows1, sem1).wait()
      pltpu.sync_copy(rows1, agg_s.at[dst_v.at[r + 1]], add=True)

      @pl.when(r < nrows - 2)
      def _():
        pltpu.async_copy(h_hbm.at[src_v.at[r + 3]], rows1, sem1)

  _batch(s * ROWS_C, HROWS)
  _batch(s * ROWS_C + HROWS, HROWS)

  plsc.subcore_barrier()

  @pl.when(c == 0)
  def _():
    pltpu.sync_copy(agg_s.at[pl.ds(s * OUT_RPT, OUT_RPT)],
                    agg0.at[pl.ds(s * OUT_RPT, OUT_RPT)])

  @pl.when(c == 1)
  def _():
    pltpu.sync_copy(agg_s.at[pl.ds(s * OUT_RPT, OUT_RPT)],
                    agg1.at[pl.ds(s * OUT_RPT, OUT_RPT)])


_agg_call = pl.kernel(
    _agg_body,
    out_type=[
        jax.ShapeDtypeStruct((NP, 128), jnp.float32),
        jax.ShapeDtypeStruct((NP, 128), jnp.float32),
    ],
    mesh=_MESH,
    scratch_types=[
        pltpu.VMEM((HROWS, 128), jnp.int32),
        pltpu.VMEM((HROWS, 128), jnp.int32),
        pltpu.VMEM((128, 128), jnp.float32),
        pltpu.VMEM((128, 128), jnp.float32),
        pltpu.VMEM_SHARED((NP, 128), jnp.float32),
        pltpu.SemaphoreType.DMA,
        pltpu.SemaphoreType.DMA,
    ],
    compiler_params=_SC_PARAMS,
)


def _prep_body(do0_ref, do1_ref, di0_ref, di1_ref, feats_ref, h_ref, nd_ref,
               ns_scr, nd_scr):
  i = pl.program_id(0)

  @pl.when(i == 0)
  def _():
    ones = jnp.ones((NC, 1), jnp.float32)
    cd = (((0,), (0,)), ((), ()))
    dop = jnp.stack([do0_ref[...], do1_ref[...]])
    dip = jnp.stack([di0_ref[...], di1_ref[...]])
    deg_o = lax.dot_general(dop, ones, cd,
                            preferred_element_type=jnp.float32)
    deg_i = lax.dot_general(dip, ones, cd,
                            preferred_element_type=jnp.float32)
    ns_scr[...] = jnp.where(deg_o > 0, lax.rsqrt(jnp.maximum(deg_o, 1.0)),
                            0.0)
    nd_scr[...] = jnp.where(deg_i > 0, lax.rsqrt(jnp.maximum(deg_i, 1.0)),
                            0.0)

  ns_blk = ns_scr[pl.ds(i * _BR, _BR), :]
  nd_ref[...] = nd_scr[pl.ds(i * _BR, _BR), :]
  h = feats_ref[...] * ns_blk
  h_ref[0] = h[:, :HALF]
  h_ref[1] = h[:, HALF:]


def _res_body(feats_ref, rw_ref, b_ref, rb_ref, yres_ref):
  yres = lax.dot_general(feats_ref[...], rw_ref[...],
                         (((1,), (1,)), ((), ())),
                         preferred_element_type=jnp.float32)
  yres_ref[...] = yres + b_ref[...] + rb_ref[...]


def _mm_body(agg0_ref, agg1_ref, nd_ref, yres_ref, w_ref, g_ref, bt_ref,
             out_ref, y_scr, sums_scr):
  i = pl.program_id(0)
  nb = N // _BR

  @pl.when(i < nb)
  def _():
    nd = nd_ref[...]
    w = w_ref[...]
    y = jnp.dot(agg0_ref[...] * nd, w[:HALF, :],
                preferred_element_type=jnp.float32)
    y = y + jnp.dot(agg1_ref[...] * nd, w[HALF:, :],
                    preferred_element_type=jnp.float32)
    y = y + yres_ref[...]
    y_scr[pl.ds(i * _BR, _BR), :] = y
    part = jnp.concatenate(
        [jnp.sum(y, axis=0, keepdims=True),
         jnp.sum(y * y, axis=0, keepdims=True)], axis=0)

    @pl.when(i == 0)
    def _():
      sums_scr[...] = part

    @pl.when(i > 0)
    def _():
      sums_scr[...] = sums_scr[...] + part

  @pl.when(i >= nb)
  def _():
    y = y_scr[pl.ds((i - nb) * _BR, _BR), :]
    mean = sums_scr[0:1, :] * (1.0 / N)
    var = sums_scr[1:2, :] * (1.0 / N) - mean * mean
    scale = lax.rsqrt(var + 1e-5) * g_ref[...]
    out_ref[...] = (y - mean) * scale + bt_ref[...]


_BR = 5000


def kernel(feats, edge_index, W, b, res_W, res_b, bn_gamma, bn_beta):
  pad = ((jnp.arange(EPAD, dtype=jnp.int32) % (NP - N)) + N)[None, :]
  ei3 = jnp.concatenate(
      [edge_index.astype(jnp.int32),
       jnp.broadcast_to(pad, (2, EPAD))], axis=1).reshape(2, EROWS, 128)

  dop, dip = _deg_call(ei3)

  h, nd = pl.pallas_call(
      _prep_body,
      grid=(N // _BR,),
      in_specs=[
          pl.BlockSpec((NP,), lambda i: (0,)),
          pl.BlockSpec((NP,), lambda i: (1,)),
          pl.BlockSpec((NP,), lambda i: (0,)),
          pl.BlockSpec((NP,), lambda i: (1,)),
          pl.BlockSpec((_BR, D), lambda i: (i, 0)),
      ],
      out_specs=[
          pl.BlockSpec((NC, _BR, HALF), lambda i: (0, i, 0)),
          pl.BlockSpec((_BR, 1), lambda i: (i, 0)),
      ],
      out_shape=[
          jax.ShapeDtypeStruct((NC, NP, HALF), jnp.float32),
          jax.ShapeDtypeStruct((N, 1), jnp.float32),
      ],
      scratch_shapes=[
          pltpu.VMEM((NP, 1), jnp.float32),
          pltpu.VMEM((NP, 1), jnp.float32),
      ],
  )(dop, dop, dip, dip, feats)
  h_flat = h.reshape(NC * NP, HALF)

  nb = N // _BR
  const = lambda i: (0, 0)

  yres = pl.pallas_call(
      _res_body,
      grid=(nb,),
      in_specs=[
          pl.BlockSpec((_BR, D), lambda i: (i, 0)),
          pl.BlockSpec((D, D), const),
          pl.BlockSpec((1, D), const),
          pl.BlockSpec((1, D), const),
      ],
      out_specs=pl.BlockSpec((_BR, D), lambda i: (i, 0)),
      out_shape=jax.ShapeDtypeStruct((N, D), jnp.float32),
  )(feats, res_W, b.reshape(1, D), res_b.reshape(1, D))

  agg0, agg1 = _agg_call(h_flat, ei3)

  phased = lambda i: (jnp.minimum(i, nb - 1), 0)
  out = pl.pallas_call(
      _mm_body,
      grid=(2 * nb,),
      in_specs=[
          pl.BlockSpec((_BR, HALF), phased),
          pl.BlockSpec((_BR, HALF), phased),
          pl.BlockSpec((_BR, 1), phased),
          pl.BlockSpec((_BR, D), phased),
          pl.BlockSpec((D, D), const),
          pl.BlockSpec((1, D), const),
          pl.BlockSpec((1, D), const),
      ],
      out_specs=pl.BlockSpec((_BR, D), lambda i: (jnp.maximum(i - nb, 0), 0)),
      out_shape=jax.ShapeDtypeStruct((N, D), jnp.float32),
      scratch_shapes=[
          pltpu.VMEM((N, D), jnp.float32),
          pltpu.VMEM((2, D), jnp.float32),
      ],
  )(agg0, agg1, nd, yres, W, bn_gamma.reshape(1, D), bn_beta.reshape(1, D))
  return out


# speedup vs baseline: 1.0093x; 1.0093x over previous
import functools

import jax
import jax.numpy as jnp
from jax import lax
from jax.experimental import pallas as pl
from jax.experimental.pallas import tpu as pltpu
from jax.experimental.pallas import tpu_sc as plsc

N = 10000
E = 160000
D = 256
HALF = 128
NC = 2
NS = 16
NP = 10240
EROWS = 1280
EPAD = EROWS * 128 - E
ROWS_A = EROWS // (NC * NS)
ROWS_C = EROWS // NS
OUT_RPT = NP // NS

_MESH = plsc.VectorSubcoreMesh(
    core_axis_name="c", subcore_axis_name="s", num_cores=NC, num_subcores=NS
)
_SC_PARAMS = pltpu.CompilerParams(needs_layout_passes=False)


SEG = 2 * NP // NS


def _deg_body(ei3, out_o, out_i, idx_s, idx_d, hist_o, hist_i, red_v,
              stage_s):
  c = lax.axis_index("c")
  s = lax.axis_index("s")
  wid = s * NC + c
  zero = jnp.zeros((16,), jnp.float32)

  @pl.loop(0, NP // 16)
  def _(i):
    hist_o[pl.ds(i * 16, 16)] = zero
    hist_i[pl.ds(i * 16, 16)] = zero

  ones = jnp.ones((16,), jnp.float32)

  pltpu.sync_copy(ei3.at[0, pl.ds(wid * ROWS_A, ROWS_A)], idx_s)
  pltpu.sync_copy(ei3.at[1, pl.ds(wid * ROWS_A, ROWS_A)], idx_d)

  @pl.loop(0, ROWS_A)
  def _(r):
    for k in range(8):
      vs = idx_s[r, pl.ds(k * 16, 16)]
      vd = idx_d[r, pl.ds(k * 16, 16)]
      plsc.addupdate_scatter(hist_o, [vs], ones)
      plsc.addupdate_scatter(hist_i, [vd], ones)

  pltpu.sync_copy(hist_o, stage_s.at[pl.ds((2 * s) * NP, NP)])
  pltpu.sync_copy(hist_i, stage_s.at[pl.ds((2 * s + 1) * NP, NP)])
  plsc.subcore_barrier()
  for t in range(NS):
    pltpu.sync_copy(stage_s.at[pl.ds(t * 2 * NP + s * SEG, SEG)], red_v.at[t])

  @pl.loop(0, SEG // 16)
  def _(j):
    acc = red_v[0, pl.ds(j * 16, 16)]
    for t in range(1, NS):
      acc = acc + red_v[t, pl.ds(j * 16, 16)]
    hist_o[pl.ds(j * 16, 16)] = acc

  @pl.when(s < 8)
  def _():
    pltpu.sync_copy(hist_o.at[pl.ds(0, SEG)],
                    out_o.at[pl.ds(c * NP + s * SEG, SEG)])

  @pl.when(s >= 8)
  def _():
    pltpu.sync_copy(hist_o.at[pl.ds(0, SEG)],
                    out_i.at[pl.ds(c * NP + (s - 8) * SEG, SEG)])


_deg_call = pl.kernel(
    _deg_body,
    out_type=[
        jax.ShapeDtypeStruct((NC * NP,), jnp.float32),
        jax.ShapeDtypeStruct((NC * NP,), jnp.float32),
    ],
    mesh=_MESH,
    scratch_types=[
        pltpu.VMEM((ROWS_A, 128), jnp.int32),
        pltpu.VMEM((ROWS_A, 128), jnp.int32),
        pltpu.VMEM((NP,), jnp.float32),
        pltpu.VMEM((NP,), jnp.float32),
        pltpu.VMEM((NS, SEG), jnp.float32),
        pltpu.VMEM_SHARED((NS * 2 * NP,), jnp.float32),
    ],
    compiler_params=_SC_PARAMS,
)


HROWS = ROWS_C // 2


def _agg_body(h_hbm, ei3, agg0, agg1, src_v, dst_v, rows0, rows1,
              agg_s, sem0, sem1):
  c = lax.axis_index("c")
  s = lax.axis_index("s")
  zero = jnp.zeros((16,), jnp.float32)

  @pl.loop(0, 128)
  def _(r):
    for k in range(8):
      rows0[r, pl.ds(k * 16, 16)] = zero

  for j in range(OUT_RPT // 128):
    pltpu.sync_copy(rows0, agg_s.at[pl.ds(s * OUT_RPT + j * 128, 128)])
  plsc.subcore_barrier()

  offv = jnp.broadcast_to(c * NP, (16,)).astype(jnp.int32)

  def _batch(base, nrows):
    pltpu.sync_copy(ei3.at[0, pl.ds(base, nrows)], src_v.at[pl.ds(0, nrows)])
    pltpu.sync_copy(ei3.at[1, pl.ds(base, nrows)], dst_v.at[pl.ds(0, nrows)])

    @pl.loop(0, nrows)
    def _(r):
      for k in range(8):
        src_v[r, pl.ds(k * 16, 16)] = src_v[r, pl.ds(k * 16, 16)] + offv

    pltpu.async_copy(h_hbm.at[src_v.at[0]], rows0, sem0)
    pltpu.async_copy(h_hbm.at[src_v.at[1]], rows1, sem1)

    @pl.loop(0, nrows, step=2)
    def _(r):
      pltpu.make_async_copy(h_hbm.at[src_v.at[0]], rows0, sem0).wait()
      pltpu.sync_copy(rows0, agg_s.at[dst_v.at[r]], add=True)

      @pl.when(r < nrows - 2)
      def _():
        pltpu.async_copy(h_hbm.at[src_v.at[r + 2]], rows0, sem0)

      pltpu.make_async_copy(h_hbm.at[src_v.at[1]], rows1, sem1).wait()
      pltpu.sync_copy(rows1, agg_s.at[dst_v.at[r + 1]], add=True)

      @pl.when(r < nrows - 2)
      def _():
        pltpu.async_copy(h_hbm.at[src_v.at[r + 3]], rows1, sem1)

  _batch(s * ROWS_C, HROWS)
  _batch(s * ROWS_C + HROWS, HROWS)

  plsc.subcore_barrier()

  @pl.when(c == 0)
  def _():
    pltpu.sync_copy(agg_s.at[pl.ds(s * OUT_RPT, OUT_RPT)],
                    agg0.at[pl.ds(s * OUT_RPT, OUT_RPT)])

  @pl.when(c == 1)
  def _():
    pltpu.sync_copy(agg_s.at[pl.ds(s * OUT_RPT, OUT_RPT)],
                    agg1.at[pl.ds(s * OUT_RPT, OUT_RPT)])


_agg_call = pl.kernel(
    _agg_body,
    out_type=[
        jax.ShapeDtypeStruct((NP, 128), jnp.float32),
        jax.ShapeDtypeStruct((NP, 128), jnp.float32),
    ],
    mesh=_MESH,
    scratch_types=[
        pltpu.VMEM((HROWS, 128), jnp.int32),
        pltpu.VMEM((HROWS, 128), jnp.int32),
        pltpu.VMEM((128, 128), jnp.float32),
        pltpu.VMEM((128, 128), jnp.float32),
        pltpu.VMEM_SHARED((NP, 128), jnp.float32),
        pltpu.SemaphoreType.DMA,
        pltpu.SemaphoreType.DMA,
    ],
    compiler_params=_SC_PARAMS,
)


def _prep_body(do0_ref, do1_ref, di0_ref, di1_ref, feats_ref, h_ref, nd_ref,
               ns_scr, nd_scr):
  i = pl.program_id(0)

  @pl.when(i == 0)
  def _():
    ones = jnp.ones((NC, 1), jnp.float32)
    cd = (((0,), (0,)), ((), ()))
    dop = jnp.stack([do0_ref[...], do1_ref[...]])
    dip = jnp.stack([di0_ref[...], di1_ref[...]])
    deg_o = lax.dot_general(dop, ones, cd,
                            preferred_element_type=jnp.float32)
    deg_i = lax.dot_general(dip, ones, cd,
                            preferred_element_type=jnp.float32)
    ns_scr[...] = jnp.where(deg_o > 0, lax.rsqrt(jnp.maximum(deg_o, 1.0)),
                            0.0)
    nd_scr[...] = jnp.where(deg_i > 0, lax.rsqrt(jnp.maximum(deg_i, 1.0)),
                            0.0)

  ns_blk = ns_scr[pl.ds(i * _BR, _BR), :]
  nd_ref[...] = nd_scr[pl.ds(i * _BR, _BR), :]
  h = feats_ref[...] * ns_blk
  h_ref[0] = h[:, :HALF]
  h_ref[1] = h[:, HALF:]


def _mm_body(agg0_ref, agg1_ref, nd_ref, feats_ref, w_ref, rw_ref, b_ref,
             rb_ref, g_ref, bt_ref, out_ref, y_scr, sums_scr):
  i = pl.program_id(0)
  nb = N // _BR

  @pl.when(i < nb)
  def _():
    nd = nd_ref[...]
    w = w_ref[...]
    y = jnp.dot(agg0_ref[...] * nd, w[:HALF, :],
                preferred_element_type=jnp.float32)
    y = y + jnp.dot(agg1_ref[...] * nd, w[HALF:, :],
                    preferred_element_type=jnp.float32)
    y = y + lax.dot_general(feats_ref[...], rw_ref[...],
                            (((1,), (1,)), ((), ())),
                            preferred_element_type=jnp.float32)
    y = y + b_ref[...] + rb_ref[...]
    y_scr[pl.ds(i * _BR, _BR), :] = y
    part = jnp.concatenate(
        [jnp.sum(y, axis=0, keepdims=True),
         jnp.sum(y * y, axis=0, keepdims=True)], axis=0)

    @pl.when(i == 0)
    def _():
      sums_scr[...] = part

    @pl.when(i > 0)
    def _():
      sums_scr[...] = sums_scr[...] + part

  @pl.when(i >= nb)
  def _():
    y = y_scr[pl.ds((i - nb) * _BR, _BR), :]
    mean = sums_scr[0:1, :] * (1.0 / N)
    var = sums_scr[1:2, :] * (1.0 / N) - mean * mean
    scale = lax.rsqrt(var + 1e-5) * g_ref[...]
    out_ref[...] = (y - mean) * scale + bt_ref[...]


_BR = 5000


def kernel(feats, edge_index, W, b, res_W, res_b, bn_gamma, bn_beta):
  pad = ((jnp.arange(EPAD, dtype=jnp.int32) % (NP - N)) + N)[None, :]
  ei3 = jnp.concatenate(
      [edge_index.astype(jnp.int32),
       jnp.broadcast_to(pad, (2, EPAD))], axis=1).reshape(2, EROWS, 128)

  dop, dip = _deg_call(ei3)

  h, nd = pl.pallas_call(
      _prep_body,
      grid=(N // _BR,),
      in_specs=[
          pl.BlockSpec((NP,), lambda i: (0,)),
          pl.BlockSpec((NP,), lambda i: (1,)),
          pl.BlockSpec((NP,), lambda i: (0,)),
          pl.BlockSpec((NP,), lambda i: (1,)),
          pl.BlockSpec((_BR, D), lambda i: (i, 0)),
      ],
      out_specs=[
          pl.BlockSpec((NC, _BR, HALF), lambda i: (0, i, 0)),
          pl.BlockSpec((_BR, 1), lambda i: (i, 0)),
      ],
      out_shape=[
          jax.ShapeDtypeStruct((NC, NP, HALF), jnp.float32),
          jax.ShapeDtypeStruct((N, 1), jnp.float32),
      ],
      scratch_shapes=[
          pltpu.VMEM((NP, 1), jnp.float32),
          pltpu.VMEM((NP, 1), jnp.float32),
      ],
  )(dop, dop, dip, dip, feats)
  h_flat = h.reshape(NC * NP, HALF)

  agg0, agg1 = _agg_call(h_flat, ei3)

  nb = N // _BR
  phased = lambda i: (jnp.minimum(i, nb - 1), 0)
  const = lambda i: (0, 0)
  out = pl.pallas_call(
      _mm_body,
      grid=(2 * nb,),
      in_specs=[
          pl.BlockSpec((_BR, HALF), phased),
          pl.BlockSpec((_BR, HALF), phased),
          pl.BlockSpec((_BR, 1), phased),
          pl.BlockSpec((_BR, D), phased),
          pl.BlockSpec((D, D), const),
          pl.BlockSpec((D, D), const),
          pl.BlockSpec((1, D), const),
          pl.BlockSpec((1, D), const),
          pl.BlockSpec((1, D), const),
          pl.BlockSpec((1, D), const),
      ],
      out_specs=pl.BlockSpec((_BR, D), lambda i: (jnp.maximum(i - nb, 0), 0)),
      out_shape=jax.ShapeDtypeStruct((N, D), jnp.float32),
      scratch_shapes=[
          pltpu.VMEM((N, D), jnp.float32),
          pltpu.VMEM((2, D), jnp.float32),
      ],
  )(agg0, agg1, nd, feats, W, res_W, b.reshape(1, D), res_b.reshape(1, D),
    bn_gamma.reshape(1, D), bn_beta.reshape(1, D))
  return out
